# Initial kernel scaffold; baseline (speedup 1.0000x reference)
import jax
import jax.numpy as jnp
from jax.experimental import pallas as pl


def kernel(x, edge_index, W1, b1, W2, b2):
    raise NotImplementedError("write your pallas kernel here")


# speedup vs baseline: 20.4788x; 20.4788x over previous
import functools

import jax
import jax.numpy as jnp
from jax import lax
from jax.experimental import pallas as pl
from jax.experimental.pallas import tpu as pltpu
from jax.experimental.pallas import tpu_sc as plsc

N = 10000
E = 320000
D = 128
C = 4
CP = 16

NC, NS = 2, 16
NW = NC * NS
CHUNK = 128
CPW = 79
EPW = CPW * CHUNK
EP = NW * EPW
NP = 10240
RPS = NP // NS

_mesh = plsc.VectorSubcoreMesh(core_axis_name="c", subcore_axis_name="s",
                               num_cores=NC, num_subcores=NS)


def _wid():
    return lax.axis_index("s") * NC + lax.axis_index("c")


@functools.partial(
    pl.kernel,
    out_type=(jax.ShapeDtypeStruct((NP,), jnp.float32),
              jax.ShapeDtypeStruct((NP,), jnp.float32)),
    mesh=_mesh,
    scratch_types=[
        pltpu.VMEM((CPW, CHUNK), jnp.int32),
        pltpu.VMEM((CHUNK,), jnp.float32),
        pltpu.VMEM((RPS,), jnp.float32),
        pltpu.VMEM_SHARED((NP,), jnp.float32),
    ],
)
def _deg_sc(dst_hbm, out0, out1, dst_v, ones_v, zv, accum):
    cid = lax.axis_index("c")
    sid = lax.axis_index("s")
    wid = _wid()
    for i in range(CHUNK // 16):
        ones_v[pl.ds(i * 16, 16)] = jnp.ones((16,), jnp.float32)
    for i in range(RPS // 16):
        zv[pl.ds(i * 16, 16)] = jnp.zeros((16,), jnp.float32)
    pltpu.sync_copy(zv, accum.at[pl.ds(sid * RPS, RPS)])
    plsc.subcore_barrier()
    pltpu.sync_copy(dst_hbm.at[wid], dst_v)

    @pl.loop(0, CPW)
    def _(c):
        pltpu.sync_copy(ones_v, accum.at[dst_v.at[c]], add=True)

    plsc.subcore_barrier()
    sl = pl.ds(sid * RPS, RPS)

    @pl.when(cid == 0)
    def _():
        pltpu.sync_copy(accum.at[sl], out0.at[sl])

    @pl.when(cid == 1)
    def _():
        pltpu.sync_copy(accum.at[sl], out1.at[sl])


def _make_scatter(width):
    @functools.partial(
        pl.kernel,
        out_type=(jax.ShapeDtypeStruct((NP, width), jnp.float32),
                  jax.ShapeDtypeStruct((NP, width), jnp.float32)),
        mesh=_mesh,
        scratch_types=[
            pltpu.VMEM((CPW, CHUNK), jnp.int32),
            pltpu.VMEM((CPW, CHUNK), jnp.int32),
            pltpu.VMEM((CHUNK, width), jnp.float32),
            pltpu.VMEM((16, width), jnp.float32),
            pltpu.VMEM_SHARED((NP, width), jnp.float32),
            pltpu.SemaphoreType.DMA,
        ],
        compiler_params=pltpu.CompilerParams(use_tc_tiling_on_sc=False),
    )
    def _scatter(src_hbm, dst_hbm, tab_hbm, out0, out1,
                 src_v, dst_v, rows_v, zv, accum, sem):
        cid = lax.axis_index("c")
        sid = lax.axis_index("s")
        wid = _wid()
        for r in range(16):
            for c in range(width // 16):
                zv[r, pl.ds(c * 16, 16)] = jnp.zeros((16,), jnp.float32)

        @pl.loop(0, RPS // 16)
        def _(j):
            pltpu.sync_copy(zv, accum.at[pl.ds(sid * RPS + j * 16, 16)])

        plsc.subcore_barrier()
        pltpu.sync_copy(src_hbm.at[wid], src_v)
        pltpu.sync_copy(dst_hbm.at[wid], dst_v)

        @pl.loop(0, CPW)
        def _(c):
            pltpu.async_copy(tab_hbm.at[src_v.at[c]], rows_v, sem).wait()
            pltpu.sync_copy(rows_v, accum.at[dst_v.at[c]], add=True)

        plsc.subcore_barrier()
        sl = pl.ds(sid * RPS, RPS)

        @pl.when(cid == 0)
        def _():
            pltpu.sync_copy(accum.at[sl], out0.at[sl])

        @pl.when(cid == 1)
        def _():
            pltpu.sync_copy(accum.at[sl], out1.at[sl])

    return _scatter


_scatter_wide = _make_scatter(D)
_scatter_thin = _make_scatter(CP)


_BLK = 1024


def _rsqrt_deg(dp_ref):
    return lax.rsqrt(dp_ref[0] + dp_ref[1] + 1.0)


def _mm_scale_body(x_ref, w_ref, dp_ref, o_ref):
    ds = _rsqrt_deg(dp_ref)
    o_ref[...] = jnp.dot(x_ref[...], w_ref[...],
                         preferred_element_type=jnp.float32) * ds[:, None]


def _mid_body(sa_ref, sb_ref, gs_ref, dp_ref, b1_ref, w2_ref, o_ref):
    ds = _rsqrt_deg(dp_ref)
    t = (sa_ref[...] + sb_ref[...] + gs_ref[...]) * ds[:, None] + b1_ref[...]
    h = jnp.maximum(t, 0.0)
    o_ref[...] = jnp.dot(h, w2_ref[...],
                         preferred_element_type=jnp.float32) * ds[:, None]


def _final_body(sa_ref, sb_ref, fs_ref, dp_ref, b2_ref, o_ref):
    ds = _rsqrt_deg(dp_ref)
    o_ref[...] = (sa_ref[...] + sb_ref[...] + fs_ref[...]) * ds[:, None] \
        + b2_ref[...]


def _row_spec(w):
    return pl.BlockSpec((_BLK, w), lambda i: (i, 0))


def _const_spec(shape):
    return pl.BlockSpec(shape, lambda i: (0,) * len(shape))


_DP_SPEC = pl.BlockSpec((2, _BLK), lambda i: (0, i))

_mm_scale_tc = pl.pallas_call(
    _mm_scale_body,
    grid=(NP // _BLK,),
    in_specs=[_row_spec(D), _const_spec((D, D)), _DP_SPEC],
    out_specs=_row_spec(D),
    out_shape=jax.ShapeDtypeStruct((NP, D), jnp.float32),
)

_mid_tc = pl.pallas_call(
    _mid_body,
    grid=(NP // _BLK,),
    in_specs=[_row_spec(D), _row_spec(D), _row_spec(D), _DP_SPEC,
              _const_spec((1, D)), _const_spec((D, CP))],
    out_specs=_row_spec(CP),
    out_shape=jax.ShapeDtypeStruct((NP, CP), jnp.float32),
)

_final_tc = pl.pallas_call(
    _final_body,
    grid=(NP // _BLK,),
    in_specs=[_row_spec(CP), _row_spec(CP), _row_spec(CP), _DP_SPEC,
              _const_spec((1, CP))],
    out_specs=_row_spec(CP),
    out_shape=jax.ShapeDtypeStruct((NP, CP), jnp.float32),
)


def kernel(x, edge_index, W1, b1, W2, b2):
    src = edge_index[0]
    dst = edge_index[1]
    pad = EP - E
    src_p = jnp.concatenate([src, jnp.full((pad,), N, jnp.int32)])
    dst_p = jnp.concatenate([dst, jnp.full((pad,), N, jnp.int32)])
    src_p = src_p.reshape(NW, CPW, CHUNK)
    dst_p = dst_p.reshape(NW, CPW, CHUNK)

    x_p = jnp.pad(x, ((0, NP - N), (0, 0)))
    w2_p = jnp.pad(W2, ((0, 0), (0, CP - C)))
    b2_p = jnp.pad(b2, (0, CP - C))

    deg0, deg1 = _deg_sc(dst_p)
    dp = jnp.stack([deg0, deg1])

    gs = _mm_scale_tc(x_p, W1, dp)
    s1a, s1b = _scatter_wide(src_p, dst_p, gs)
    fs = _mid_tc(s1a, s1b, gs, dp, b1.reshape(1, D), w2_p)
    s2a, s2b = _scatter_thin(src_p, dst_p, fs)
    outp = _final_tc(s2a, s2b, fs, dp, b2_p.reshape(1, CP))
    return outp[:N, :C]
